# baseline (speedup 1.0000x reference)
import functools

import jax
import jax.numpy as jnp
from jax import lax
from jax.experimental import pallas as pl
from jax.experimental.pallas import tpu as pltpu
from jax.experimental.pallas import tpu_sc as plsc

N = 10000
E = 320000
D = 128

_NC = 2
_NS = 16
_NW = _NC * _NS
_R = E // _NW
_S = 80
_NSTR = _R // _S
_SLOTS = 5
_ZR = N // _NS
_SLOTS2 = 3
_K = 5
_EC = E // _K
_RC = _R // _K
_NSTRC = _RC // _S

@functools.cache
def _mesh():
  return plsc.VectorSubcoreMesh(core_axis_name="c", subcore_axis_name="s",
                                num_cores=_NC, num_subcores=_NS)


_GSLOTS = 3


def _gather_body(ts_hbm, tr_hbm, idxs_hbm, idxr_hbm, out_s, out_r,
                 idx_v, buf, gsem, tab_sh):
  cid = lax.axis_index("c")
  sid = lax.axis_index("s")
  wid = sid * _NC + cid
  row0 = wid * _RC
  zr8 = (_ZR // 8) * 8

  main = _NSTRC - (_NSTRC % _GSLOTS)

  for tab_hbm, idx_hbm, out in ((ts_hbm, idxs_hbm, out_s),
                                (tr_hbm, idxr_hbm, out_r)):
    pltpu.sync_copy(tab_hbm.at[pl.ds(sid * zr8, zr8)],
                    tab_sh.at[pl.ds(sid * zr8, zr8)])

    @pl.when(sid == 0)
    def _():
      rem = N - _NS * zr8
      pltpu.sync_copy(tab_hbm.at[pl.ds(_NS * zr8, rem)],
                      tab_sh.at[pl.ds(_NS * zr8, rem)])

    pltpu.sync_copy(idx_hbm.at[pl.ds(row0, _RC)], idx_v)
    plsc.subcore_barrier()

    def fire(g, b):
      pltpu.async_copy(tab_sh.at[idx_v.at[pl.ds(g * _S, _S)]], buf.at[b],
                       gsem.at[b])

    def consume(g, b):
      pltpu.make_async_copy(tab_sh.at[idx_v.at[pl.ds(g * _S, _S)]],
                            buf.at[b], gsem.at[b]).wait()
      pltpu.sync_copy(buf.at[b], out.at[pl.ds(row0 + g * _S, _S)])

    for b in range(_GSLOTS):
      fire(b, b)

    @pl.loop(0, main, step=_GSLOTS)
    def _(g0):
      for b in range(_GSLOTS):
        g = g0 + b
        consume(g, b)

        @pl.when(g + _GSLOTS < _NSTRC)
        def _():
          fire(g + _GSLOTS, b)

    for t in range(main, _NSTRC):
      consume(t, t - main)

    plsc.subcore_barrier()


def _sc_gather(snf, rnf, snd, rcv):
  return pl.kernel(
      _gather_body,
      out_type=(jax.ShapeDtypeStruct((_EC, D), jnp.float32),
                jax.ShapeDtypeStruct((_EC, D), jnp.float32)),
      mesh=_mesh(),
      scratch_types=[
          pltpu.VMEM((_RC,), jnp.int32),
          pltpu.VMEM((_GSLOTS, _S, D), jnp.float32),
          pltpu.SemaphoreType.DMA((_GSLOTS,)),
          pltpu.VMEM_SHARED((N, D), jnp.float32),
      ],
  )(snf, rnf, snd, rcv)


def _scatter_body(edge_hbm, idx_hbm, zeros_hbm, out_hbm,
                  idx_v, buf, lsem, acc_sh):
  cid = lax.axis_index("c")
  sid = lax.axis_index("s")
  wid = sid * _NC + cid
  row0 = wid * _R

  pltpu.sync_copy(zeros_hbm.at[sid], acc_sh.at[pl.ds(sid * _ZR, _ZR)])
  pltpu.sync_copy(idx_hbm.at[wid], idx_v)
  plsc.subcore_barrier()

  def fire(g, b):
    pltpu.async_copy(edge_hbm.at[pl.ds(row0 + g * _S, _S)], buf.at[b],
                     lsem.at[b])

  def consume(g, b):
    pltpu.make_async_copy(edge_hbm.at[pl.ds(row0 + g * _S, _S)],
                          buf.at[b], lsem.at[b]).wait()
    pltpu.sync_copy(buf.at[b], acc_sh.at[idx_v.at[g]], add=True)

  for b in range(_SLOTS2):
    fire(b, b)

  main = _NSTR - (_NSTR % _SLOTS2)

  @pl.loop(0, main, step=_SLOTS2)
  def _(g0):
    for b in range(_SLOTS2):
      g = g0 + b
      consume(g, b)

      @pl.when(g + _SLOTS2 < _NSTR)
      def _():
        fire(g + _SLOTS2, b)

  for t in range(main, _NSTR):
    consume(t, t - main)

  plsc.subcore_barrier()
  pltpu.sync_copy(acc_sh.at[pl.ds(sid * _ZR, _ZR)], out_hbm.at[cid, sid])


def _sc_scatter(raw, rcv, zeros):
  return pl.kernel(
      _scatter_body,
      out_type=jax.ShapeDtypeStruct((_NC, _NS, _ZR, D), jnp.float32),
      mesh=_mesh(),
      scratch_types=[
          pltpu.VMEM((_NSTR, _S), jnp.int32),
          pltpu.VMEM((_SLOTS2, _S, D), jnp.float32),
          pltpu.SemaphoreType.DMA((_SLOTS2,)),
          pltpu.VMEM_SHARED((N, D), jnp.float32),
      ],
  )(raw, rcv, zeros)


_EB = 1000


def _edge_mlp_compute(gs, gr, ef, w0s, w0r, w0e, b0, w1, b1, w2, b2,
                      ln_g, ln_b, raw, oe):
  f32 = jnp.float32
  e = ef[...]
  x = (jnp.dot(gs[...], w0s[...], preferred_element_type=f32)
       + jnp.dot(gr[...], w0r[...], preferred_element_type=f32)
       + jnp.dot(e, w0e[...], preferred_element_type=f32) + b0[...])
  x = jnp.maximum(x, 0.0)
  x = jnp.maximum(jnp.dot(x, w1[...], preferred_element_type=f32) + b1[...],
                  0.0)
  x = jnp.dot(x, w2[...], preferred_element_type=f32) + b2[...]
  mu = jnp.mean(x, axis=1, keepdims=True)
  xc = x - mu
  var = jnp.mean(xc * xc, axis=1, keepdims=True)
  y = ln_g[...] * (xc * lax.rsqrt(var + 1e-5)) + ln_b[...]
  raw[...] = y
  oe[...] = y + e


def _edge_mlp_body0(gs, gr, ef, w0s, w0r, w0e, b0, w1, b1, w2, b2,
                    ln_g, ln_b, raw, oe):
  _edge_mlp_compute(gs, gr, ef, w0s, w0r, w0e, b0, w1, b1, w2, b2,
                    ln_g, ln_b, raw, oe)


def _edge_mlp_bodyk(gs, gr, ef, w0s, w0r, w0e, b0, w1, b1, w2, b2,
                    ln_g, ln_b, raw_in, oe_in, raw, oe):
  del raw_in, oe_in
  _edge_mlp_compute(gs, gr, ef, w0s, w0r, w0e, b0, w1, b1, w2, b2,
                    ln_g, ln_b, raw, oe)


def _edge_mlp(k, gs, gr, ef, raw_buf, oe_buf, w0s, w0r, w0e, b0, w1, b1, w2,
              b2, ln_g, ln_b):
  koff = k * (_EC // _EB)
  crow = pl.BlockSpec((_EB, D), lambda i: (i, 0))
  orow = pl.BlockSpec((_EB, D), lambda i, koff=koff: (koff + i, 0))
  mat = pl.BlockSpec((D, D), lambda i: (0, 0))
  vec = pl.BlockSpec((1, D), lambda i: (0, 0))
  any_ = pl.BlockSpec(memory_space=pl.ANY)
  wspecs = [mat, mat, mat, vec, mat, vec, mat, vec, vec, vec]
  out_shape = (jax.ShapeDtypeStruct((E, D), jnp.float32),
               jax.ShapeDtypeStruct((E, D), jnp.float32))
  args = (gs, gr, ef) + (w0s, w0r, w0e, b0, w1, b1, w2, b2, ln_g, ln_b)
  if k == 0:
    return pl.pallas_call(
        _edge_mlp_body0,
        grid=(_EC // _EB,),
        in_specs=[crow, crow, crow] + wspecs,
        out_specs=[orow, orow],
        out_shape=out_shape,
    )(*args)
  return pl.pallas_call(
      _edge_mlp_bodyk,
      grid=(_EC // _EB,),
      in_specs=[crow, crow, crow] + wspecs + [any_, any_],
      out_specs=[orow, orow],
      out_shape=out_shape,
      input_output_aliases={13: 0, 14: 1},
  )(*args, raw_buf, oe_buf)


_NB = 2000


def _node_mlp_body(rnf, p0, p1, w0r, w0a, b0, w1, b1, w2, b2, ln_g, ln_b,
                   out):
  f32 = jnp.float32
  r = rnf[...]
  acc = p0[...] + p1[...]
  x = (jnp.dot(r, w0r[...], preferred_element_type=f32)
       + jnp.dot(acc, w0a[...], preferred_element_type=f32) + b0[...])
  x = jnp.maximum(x, 0.0)
  x = jnp.maximum(jnp.dot(x, w1[...], preferred_element_type=f32) + b1[...],
                  0.0)
  x = jnp.dot(x, w2[...], preferred_element_type=f32) + b2[...]
  mu = jnp.mean(x, axis=1, keepdims=True)
  xc = x - mu
  var = jnp.mean(xc * xc, axis=1, keepdims=True)
  out[...] = ln_g[...] * (xc * lax.rsqrt(var + 1e-5)) + ln_b[...] + r


def _node_mlp(rnf, p0, p1, w0r, w0a, b0, w1, b1, w2, b2, ln_g, ln_b):
  row = pl.BlockSpec((_NB, D), lambda i: (i, 0))
  mat = pl.BlockSpec((D, D), lambda i: (0, 0))
  vec = pl.BlockSpec((1, D), lambda i: (0, 0))
  return pl.pallas_call(
      _node_mlp_body,
      grid=(N // _NB,),
      in_specs=[row, row, row, mat, mat, vec, mat, vec, mat, vec, vec, vec],
      out_specs=row,
      out_shape=jax.ShapeDtypeStruct((N, D), jnp.float32),
  )(rnf, p0, p1, w0r, w0a, b0, w1, b1, w2, b2, ln_g, ln_b)


def kernel(senders, receivers, edge_features, sender_node_features,
           receiver_node_features, We0, be0, We1, be1, We2, be2, lne_g, lne_b,
           Wn0, bn0, Wn1, bn1, Wn2, bn2, lnn_g, lnn_b):
  snd = senders[0].astype(jnp.int32)
  rcv = receivers[0].astype(jnp.int32)
  ef = edge_features[0]
  snf = sender_node_features[0]
  rnf = receiver_node_features[0]
  ew = (We0[0:D], We0[D:2 * D], We0[2 * D:3 * D], be0.reshape(1, D),
        We1, be1.reshape(1, D), We2, be2.reshape(1, D),
        lne_g.reshape(1, D), lne_b.reshape(1, D))

  raw, out_edge = None, None
  for k in range(_K):
    sl = slice(k * _EC, (k + 1) * _EC)
    gs, gr = _sc_gather(snf, rnf, snd[sl], rcv[sl])
    raw, out_edge = _edge_mlp(k, gs, gr, ef[sl], raw, out_edge, *ew)

  zeros = jnp.zeros((_NS, _ZR, D), jnp.float32)
  partials = _sc_scatter(raw, rcv.reshape(_NW, _NSTR, _S),
                         zeros).reshape(_NC, N, D)

  new_node = _node_mlp(
      rnf, partials[0], partials[1],
      Wn0[0:D], Wn0[D:2 * D], bn0.reshape(1, D),
      Wn1, bn1.reshape(1, D), Wn2, bn2.reshape(1, D),
      lnn_g.reshape(1, D), lnn_b.reshape(1, D))

  return out_edge[None], new_node[None]


# speedup vs baseline: 1.2806x; 1.2806x over previous
import functools

import jax
import jax.numpy as jnp
from jax import lax
from jax.experimental import pallas as pl
from jax.experimental.pallas import tpu as pltpu
from jax.experimental.pallas import tpu_sc as plsc

N = 10000
E = 320000
D = 128

_NC = 2
_NS = 16
_NW = _NC * _NS
_R = E // _NW
_S = 80
_NSTR = _R // _S
_SLOTS = 5
_ZR = N // _NS
_SLOTS2 = 3
_K = 1
_EC = E // _K
_RC = _R // _K
_NSTRC = _RC // _S

@functools.cache
def _mesh():
  return plsc.VectorSubcoreMesh(core_axis_name="c", subcore_axis_name="s",
                                num_cores=_NC, num_subcores=_NS)


_GSLOTS = 3


def _gather_body(ts_hbm, tr_hbm, idxs_hbm, idxr_hbm, out_s, out_r,
                 idx_v, buf, gsem, tab_sh):
  cid = lax.axis_index("c")
  sid = lax.axis_index("s")
  wid = sid * _NC + cid
  row0 = wid * _RC
  zr8 = (_ZR // 8) * 8

  main = _NSTRC - (_NSTRC % _GSLOTS)

  for tab_hbm, idx_hbm, out in ((ts_hbm, idxs_hbm, out_s),
                                (tr_hbm, idxr_hbm, out_r)):
    pltpu.sync_copy(tab_hbm.at[pl.ds(sid * zr8, zr8)],
                    tab_sh.at[pl.ds(sid * zr8, zr8)])

    @pl.when(sid == 0)
    def _():
      rem = N - _NS * zr8
      pltpu.sync_copy(tab_hbm.at[pl.ds(_NS * zr8, rem)],
                      tab_sh.at[pl.ds(_NS * zr8, rem)])

    pltpu.sync_copy(idx_hbm.at[pl.ds(row0, _RC)], idx_v)
    plsc.subcore_barrier()

    def fire(g, b):
      pltpu.async_copy(tab_sh.at[idx_v.at[pl.ds(g * _S, _S)]], buf.at[b],
                       gsem.at[b])

    def consume(g, b):
      pltpu.make_async_copy(tab_sh.at[idx_v.at[pl.ds(g * _S, _S)]],
                            buf.at[b], gsem.at[b]).wait()
      pltpu.sync_copy(buf.at[b], out.at[pl.ds(row0 + g * _S, _S)])

    for b in range(_GSLOTS):
      fire(b, b)

    @pl.loop(0, main, step=_GSLOTS)
    def _(g0):
      for b in range(_GSLOTS):
        g = g0 + b
        consume(g, b)

        @pl.when(g + _GSLOTS < _NSTRC)
        def _():
          fire(g + _GSLOTS, b)

    for t in range(main, _NSTRC):
      consume(t, t - main)

    plsc.subcore_barrier()


def _sc_gather(snf, rnf, snd, rcv):
  return pl.kernel(
      _gather_body,
      out_type=(jax.ShapeDtypeStruct((_EC, D), jnp.float32),
                jax.ShapeDtypeStruct((_EC, D), jnp.float32)),
      mesh=_mesh(),
      scratch_types=[
          pltpu.VMEM((_RC,), jnp.int32),
          pltpu.VMEM((_GSLOTS, _S, D), jnp.float32),
          pltpu.SemaphoreType.DMA((_GSLOTS,)),
          pltpu.VMEM_SHARED((N, D), jnp.float32),
      ],
  )(snf, rnf, snd, rcv)


def _scatter_body(edge_hbm, idx_hbm, zeros_hbm, out_hbm,
                  idx_v, buf, lsem, acc_sh):
  cid = lax.axis_index("c")
  sid = lax.axis_index("s")
  wid = sid * _NC + cid
  row0 = wid * _R

  pltpu.sync_copy(zeros_hbm.at[sid], acc_sh.at[pl.ds(sid * _ZR, _ZR)])
  pltpu.sync_copy(idx_hbm.at[wid], idx_v)
  plsc.subcore_barrier()

  def fire(g, b):
    pltpu.async_copy(edge_hbm.at[pl.ds(row0 + g * _S, _S)], buf.at[b],
                     lsem.at[b])

  def consume(g, b):
    pltpu.make_async_copy(edge_hbm.at[pl.ds(row0 + g * _S, _S)],
                          buf.at[b], lsem.at[b]).wait()
    pltpu.sync_copy(buf.at[b], acc_sh.at[idx_v.at[g]], add=True)

  for b in range(_SLOTS2):
    fire(b, b)

  main = _NSTR - (_NSTR % _SLOTS2)

  @pl.loop(0, main, step=_SLOTS2)
  def _(g0):
    for b in range(_SLOTS2):
      g = g0 + b
      consume(g, b)

      @pl.when(g + _SLOTS2 < _NSTR)
      def _():
        fire(g + _SLOTS2, b)

  for t in range(main, _NSTR):
    consume(t, t - main)

  plsc.subcore_barrier()
  pltpu.sync_copy(acc_sh.at[pl.ds(sid * _ZR, _ZR)], out_hbm.at[cid, sid])


def _sc_scatter(raw, rcv, zeros):
  return pl.kernel(
      _scatter_body,
      out_type=jax.ShapeDtypeStruct((_NC, _NS, _ZR, D), jnp.float32),
      mesh=_mesh(),
      scratch_types=[
          pltpu.VMEM((_NSTR, _S), jnp.int32),
          pltpu.VMEM((_SLOTS2, _S, D), jnp.float32),
          pltpu.SemaphoreType.DMA((_SLOTS2,)),
          pltpu.VMEM_SHARED((N, D), jnp.float32),
      ],
  )(raw, rcv, zeros)


_EB = 2000


def _edge_mlp_compute(gs, gr, ef, w0s, w0r, w0e, b0, w1, b1, w2, b2,
                      ln_g, ln_b, raw, oe):
  f32 = jnp.float32
  e = ef[...]
  x = (jnp.dot(gs[...], w0s[...], preferred_element_type=f32)
       + jnp.dot(gr[...], w0r[...], preferred_element_type=f32)
       + jnp.dot(e, w0e[...], preferred_element_type=f32) + b0[...])
  x = jnp.maximum(x, 0.0)
  x = jnp.maximum(jnp.dot(x, w1[...], preferred_element_type=f32) + b1[...],
                  0.0)
  x = jnp.dot(x, w2[...], preferred_element_type=f32) + b2[...]
  mu = jnp.mean(x, axis=1, keepdims=True)
  xc = x - mu
  var = jnp.mean(xc * xc, axis=1, keepdims=True)
  y = ln_g[...] * (xc * lax.rsqrt(var + 1e-5)) + ln_b[...]
  raw[...] = y
  oe[...] = y + e


def _edge_mlp_body0(gs, gr, ef, w0s, w0r, w0e, b0, w1, b1, w2, b2,
                    ln_g, ln_b, raw, oe):
  _edge_mlp_compute(gs, gr, ef, w0s, w0r, w0e, b0, w1, b1, w2, b2,
                    ln_g, ln_b, raw, oe)


def _edge_mlp_bodyk(gs, gr, ef, w0s, w0r, w0e, b0, w1, b1, w2, b2,
                    ln_g, ln_b, raw_in, oe_in, raw, oe):
  del raw_in, oe_in
  _edge_mlp_compute(gs, gr, ef, w0s, w0r, w0e, b0, w1, b1, w2, b2,
                    ln_g, ln_b, raw, oe)


def _edge_mlp(k, gs, gr, ef, raw_buf, oe_buf, w0s, w0r, w0e, b0, w1, b1, w2,
              b2, ln_g, ln_b):
  koff = k * (_EC // _EB)
  crow = pl.BlockSpec((_EB, D), lambda i: (i, 0))
  orow = pl.BlockSpec((_EB, D), lambda i, koff=koff: (koff + i, 0))
  mat = pl.BlockSpec((D, D), lambda i: (0, 0))
  vec = pl.BlockSpec((1, D), lambda i: (0, 0))
  any_ = pl.BlockSpec(memory_space=pl.ANY)
  wspecs = [mat, mat, mat, vec, mat, vec, mat, vec, vec, vec]
  out_shape = (jax.ShapeDtypeStruct((E, D), jnp.float32),
               jax.ShapeDtypeStruct((E, D), jnp.float32))
  args = (gs, gr, ef) + (w0s, w0r, w0e, b0, w1, b1, w2, b2, ln_g, ln_b)
  if k == 0:
    return pl.pallas_call(
        _edge_mlp_body0,
        grid=(_EC // _EB,),
        in_specs=[crow, crow, crow] + wspecs,
        out_specs=[orow, orow],
        out_shape=out_shape,
    )(*args)
  return pl.pallas_call(
      _edge_mlp_bodyk,
      grid=(_EC // _EB,),
      in_specs=[crow, crow, crow] + wspecs + [any_, any_],
      out_specs=[orow, orow],
      out_shape=out_shape,
      input_output_aliases={13: 0, 14: 1},
  )(*args, raw_buf, oe_buf)


_NB = 2000


def _node_mlp_body(rnf, p0, p1, w0r, w0a, b0, w1, b1, w2, b2, ln_g, ln_b,
                   out):
  f32 = jnp.float32
  r = rnf[...]
  acc = p0[...] + p1[...]
  x = (jnp.dot(r, w0r[...], preferred_element_type=f32)
       + jnp.dot(acc, w0a[...], preferred_element_type=f32) + b0[...])
  x = jnp.maximum(x, 0.0)
  x = jnp.maximum(jnp.dot(x, w1[...], preferred_element_type=f32) + b1[...],
                  0.0)
  x = jnp.dot(x, w2[...], preferred_element_type=f32) + b2[...]
  mu = jnp.mean(x, axis=1, keepdims=True)
  xc = x - mu
  var = jnp.mean(xc * xc, axis=1, keepdims=True)
  out[...] = ln_g[...] * (xc * lax.rsqrt(var + 1e-5)) + ln_b[...] + r


def _node_mlp(rnf, p0, p1, w0r, w0a, b0, w1, b1, w2, b2, ln_g, ln_b):
  row = pl.BlockSpec((_NB, D), lambda i: (i, 0))
  mat = pl.BlockSpec((D, D), lambda i: (0, 0))
  vec = pl.BlockSpec((1, D), lambda i: (0, 0))
  return pl.pallas_call(
      _node_mlp_body,
      grid=(N // _NB,),
      in_specs=[row, row, row, mat, mat, vec, mat, vec, mat, vec, vec, vec],
      out_specs=row,
      out_shape=jax.ShapeDtypeStruct((N, D), jnp.float32),
  )(rnf, p0, p1, w0r, w0a, b0, w1, b1, w2, b2, ln_g, ln_b)


def kernel(senders, receivers, edge_features, sender_node_features,
           receiver_node_features, We0, be0, We1, be1, We2, be2, lne_g, lne_b,
           Wn0, bn0, Wn1, bn1, Wn2, bn2, lnn_g, lnn_b):
  snd = senders[0].astype(jnp.int32)
  rcv = receivers[0].astype(jnp.int32)
  ef = edge_features[0]
  snf = sender_node_features[0]
  rnf = receiver_node_features[0]
  ew = (We0[0:D], We0[D:2 * D], We0[2 * D:3 * D], be0.reshape(1, D),
        We1, be1.reshape(1, D), We2, be2.reshape(1, D),
        lne_g.reshape(1, D), lne_b.reshape(1, D))

  raw, out_edge = None, None
  for k in range(_K):
    sl = slice(k * _EC, (k + 1) * _EC)
    gs, gr = _sc_gather(snf, rnf, snd[sl], rcv[sl])
    raw, out_edge = _edge_mlp(k, gs, gr, ef[sl], raw, out_edge, *ew)

  zeros = jnp.zeros((_NS, _ZR, D), jnp.float32)
  partials = _sc_scatter(raw, rcv.reshape(_NW, _NSTR, _S),
                         zeros).reshape(_NC, N, D)

  new_node = _node_mlp(
      rnf, partials[0], partials[1],
      Wn0[0:D], Wn0[D:2 * D], bn0.reshape(1, D),
      Wn1, bn1.reshape(1, D), Wn2, bn2.reshape(1, D),
      lnn_g.reshape(1, D), lnn_b.reshape(1, D))

  return out_edge[None], new_node[None]


# speedup vs baseline: 1.4129x; 1.1032x over previous
import functools

import jax
import jax.numpy as jnp
from jax import lax
from jax.experimental import pallas as pl
from jax.experimental.pallas import tpu as pltpu
from jax.experimental.pallas import tpu_sc as plsc

N = 10000
E = 320000
D = 128

_NC = 2
_NS = 16
_NW = _NC * _NS
_R = E // _NW
_S = 80
_NSTR = _R // _S
_SLOTS = 5
_ZR = N // _NS
_SLOTS2 = 3
_K = 1
_EC = E // _K
_RC = _R // _K
_NSTRC = _RC // _S

@functools.cache
def _mesh():
  return plsc.VectorSubcoreMesh(core_axis_name="c", subcore_axis_name="s",
                                num_cores=_NC, num_subcores=_NS)


_GSLOTS = 3


def _gather_body(ts_hbm, tr_hbm, idxs_hbm, idxr_hbm, out_s, out_r,
                 idx_v, buf, gsem, tab_sh):
  cid = lax.axis_index("c")
  sid = lax.axis_index("s")
  wid = sid * _NC + cid
  row0 = wid * _RC
  zr8 = (_ZR // 8) * 8

  main = _NSTRC - (_NSTRC % _GSLOTS)

  for tab_hbm, idx_hbm, out in ((ts_hbm, idxs_hbm, out_s),
                                (tr_hbm, idxr_hbm, out_r)):
    pltpu.sync_copy(tab_hbm.at[pl.ds(sid * zr8, zr8)],
                    tab_sh.at[pl.ds(sid * zr8, zr8)])

    @pl.when(sid == 0)
    def _():
      rem = N - _NS * zr8
      pltpu.sync_copy(tab_hbm.at[pl.ds(_NS * zr8, rem)],
                      tab_sh.at[pl.ds(_NS * zr8, rem)])

    pltpu.sync_copy(idx_hbm.at[pl.ds(row0, _RC)], idx_v)
    plsc.subcore_barrier()

    def fire(g, b):
      pltpu.async_copy(tab_sh.at[idx_v.at[pl.ds(g * _S, _S)]], buf.at[b],
                       gsem.at[b])

    def consume(g, b):
      pltpu.make_async_copy(tab_sh.at[idx_v.at[pl.ds(g * _S, _S)]],
                            buf.at[b], gsem.at[b]).wait()
      pltpu.sync_copy(buf.at[b], out.at[pl.ds(row0 + g * _S, _S)])

    for b in range(_GSLOTS):
      fire(b, b)

    @pl.loop(0, main, step=_GSLOTS)
    def _(g0):
      for b in range(_GSLOTS):
        g = g0 + b
        consume(g, b)

        @pl.when(g + _GSLOTS < _NSTRC)
        def _():
          fire(g + _GSLOTS, b)

    for t in range(main, _NSTRC):
      consume(t, t - main)

    plsc.subcore_barrier()


def _sc_gather(snf, rnf, snd, rcv):
  return pl.kernel(
      _gather_body,
      out_type=(jax.ShapeDtypeStruct((_EC, D), jnp.float32),
                jax.ShapeDtypeStruct((_EC, D), jnp.float32)),
      mesh=_mesh(),
      scratch_types=[
          pltpu.VMEM((_RC,), jnp.int32),
          pltpu.VMEM((_GSLOTS, _S, D), jnp.float32),
          pltpu.SemaphoreType.DMA((_GSLOTS,)),
          pltpu.VMEM_SHARED((N, D), jnp.float32),
      ],
  )(snf, rnf, snd, rcv)


def _scatter_body(edge_hbm, idx_hbm, zeros_hbm, out_hbm,
                  idx_v, buf, lsem, acc_sh):
  cid = lax.axis_index("c")
  sid = lax.axis_index("s")
  wid = sid * _NC + cid
  row0 = wid * _R

  pltpu.sync_copy(zeros_hbm.at[sid], acc_sh.at[pl.ds(sid * _ZR, _ZR)])
  pltpu.sync_copy(idx_hbm.at[wid], idx_v)
  plsc.subcore_barrier()

  def fire(g, b):
    pltpu.async_copy(edge_hbm.at[pl.ds(row0 + g * _S, _S)], buf.at[b],
                     lsem.at[b])

  def consume(g, b):
    pltpu.make_async_copy(edge_hbm.at[pl.ds(row0 + g * _S, _S)],
                          buf.at[b], lsem.at[b]).wait()
    pltpu.sync_copy(buf.at[b], acc_sh.at[idx_v.at[g]], add=True)

  for b in range(_SLOTS2):
    fire(b, b)

  main = _NSTR - (_NSTR % _SLOTS2)

  @pl.loop(0, main, step=_SLOTS2)
  def _(g0):
    for b in range(_SLOTS2):
      g = g0 + b
      consume(g, b)

      @pl.when(g + _SLOTS2 < _NSTR)
      def _():
        fire(g + _SLOTS2, b)

  for t in range(main, _NSTR):
    consume(t, t - main)

  plsc.subcore_barrier()
  pltpu.sync_copy(acc_sh.at[pl.ds(sid * _ZR, _ZR)], out_hbm.at[cid, sid])


def _sc_scatter(raw, rcv, zeros):
  return pl.kernel(
      _scatter_body,
      out_type=jax.ShapeDtypeStruct((_NC, _NS, _ZR, D), jnp.float32),
      mesh=_mesh(),
      scratch_types=[
          pltpu.VMEM((_NSTR, _S), jnp.int32),
          pltpu.VMEM((_SLOTS2, _S, D), jnp.float32),
          pltpu.SemaphoreType.DMA((_SLOTS2,)),
          pltpu.VMEM_SHARED((N, D), jnp.float32),
      ],
  )(raw, rcv, zeros)


_EB = 4000


def _edge_mlp_compute(gs, gr, ef, w0s, w0r, w0e, b0, w1, b1, w2, b2,
                      ln_g, ln_b, raw, oe):
  f32 = jnp.float32
  e = ef[...]
  x = (jnp.dot(gs[...], w0s[...], preferred_element_type=f32)
       + jnp.dot(gr[...], w0r[...], preferred_element_type=f32)
       + jnp.dot(e, w0e[...], preferred_element_type=f32) + b0[...])
  x = jnp.maximum(x, 0.0)
  x = jnp.maximum(jnp.dot(x, w1[...], preferred_element_type=f32) + b1[...],
                  0.0)
  x = jnp.dot(x, w2[...], preferred_element_type=f32) + b2[...]
  mu = jnp.mean(x, axis=1, keepdims=True)
  xc = x - mu
  var = jnp.mean(xc * xc, axis=1, keepdims=True)
  y = ln_g[...] * (xc * lax.rsqrt(var + 1e-5)) + ln_b[...]
  raw[...] = y
  oe[...] = y + e


def _edge_mlp_body0(gs, gr, ef, w0s, w0r, w0e, b0, w1, b1, w2, b2,
                    ln_g, ln_b, raw, oe):
  _edge_mlp_compute(gs, gr, ef, w0s, w0r, w0e, b0, w1, b1, w2, b2,
                    ln_g, ln_b, raw, oe)


def _edge_mlp_bodyk(gs, gr, ef, w0s, w0r, w0e, b0, w1, b1, w2, b2,
                    ln_g, ln_b, raw_in, oe_in, raw, oe):
  del raw_in, oe_in
  _edge_mlp_compute(gs, gr, ef, w0s, w0r, w0e, b0, w1, b1, w2, b2,
                    ln_g, ln_b, raw, oe)


def _edge_mlp(k, gs, gr, ef, raw_buf, oe_buf, w0s, w0r, w0e, b0, w1, b1, w2,
              b2, ln_g, ln_b):
  koff = k * (_EC // _EB)
  crow = pl.BlockSpec((_EB, D), lambda i: (i, 0))
  orow = pl.BlockSpec((_EB, D), lambda i, koff=koff: (koff + i, 0))
  mat = pl.BlockSpec((D, D), lambda i: (0, 0))
  vec = pl.BlockSpec((1, D), lambda i: (0, 0))
  any_ = pl.BlockSpec(memory_space=pl.ANY)
  wspecs = [mat, mat, mat, vec, mat, vec, mat, vec, vec, vec]
  out_shape = (jax.ShapeDtypeStruct((E, D), jnp.float32),
               jax.ShapeDtypeStruct((E, D), jnp.float32))
  args = (gs, gr, ef) + (w0s, w0r, w0e, b0, w1, b1, w2, b2, ln_g, ln_b)
  if k == 0:
    return pl.pallas_call(
        _edge_mlp_body0,
        grid=(_EC // _EB,),
        in_specs=[crow, crow, crow] + wspecs,
        out_specs=[orow, orow],
        out_shape=out_shape,
    )(*args)
  return pl.pallas_call(
      _edge_mlp_bodyk,
      grid=(_EC // _EB,),
      in_specs=[crow, crow, crow] + wspecs + [any_, any_],
      out_specs=[orow, orow],
      out_shape=out_shape,
      input_output_aliases={13: 0, 14: 1},
  )(*args, raw_buf, oe_buf)


_NB = 2000


def _node_mlp_body(rnf, p0, p1, w0r, w0a, b0, w1, b1, w2, b2, ln_g, ln_b,
                   out):
  f32 = jnp.float32
  r = rnf[...]
  acc = p0[...] + p1[...]
  x = (jnp.dot(r, w0r[...], preferred_element_type=f32)
       + jnp.dot(acc, w0a[...], preferred_element_type=f32) + b0[...])
  x = jnp.maximum(x, 0.0)
  x = jnp.maximum(jnp.dot(x, w1[...], preferred_element_type=f32) + b1[...],
                  0.0)
  x = jnp.dot(x, w2[...], preferred_element_type=f32) + b2[...]
  mu = jnp.mean(x, axis=1, keepdims=True)
  xc = x - mu
  var = jnp.mean(xc * xc, axis=1, keepdims=True)
  out[...] = ln_g[...] * (xc * lax.rsqrt(var + 1e-5)) + ln_b[...] + r


def _node_mlp(rnf, p0, p1, w0r, w0a, b0, w1, b1, w2, b2, ln_g, ln_b):
  row = pl.BlockSpec((_NB, D), lambda i: (i, 0))
  mat = pl.BlockSpec((D, D), lambda i: (0, 0))
  vec = pl.BlockSpec((1, D), lambda i: (0, 0))
  return pl.pallas_call(
      _node_mlp_body,
      grid=(N // _NB,),
      in_specs=[row, row, row, mat, mat, vec, mat, vec, mat, vec, vec, vec],
      out_specs=row,
      out_shape=jax.ShapeDtypeStruct((N, D), jnp.float32),
  )(rnf, p0, p1, w0r, w0a, b0, w1, b1, w2, b2, ln_g, ln_b)


def kernel(senders, receivers, edge_features, sender_node_features,
           receiver_node_features, We0, be0, We1, be1, We2, be2, lne_g, lne_b,
           Wn0, bn0, Wn1, bn1, Wn2, bn2, lnn_g, lnn_b):
  snd = senders[0].astype(jnp.int32)
  rcv = receivers[0].astype(jnp.int32)
  ef = edge_features[0]
  snf = sender_node_features[0]
  rnf = receiver_node_features[0]
  ew = (We0[0:D], We0[D:2 * D], We0[2 * D:3 * D], be0.reshape(1, D),
        We1, be1.reshape(1, D), We2, be2.reshape(1, D),
        lne_g.reshape(1, D), lne_b.reshape(1, D))

  raw, out_edge = None, None
  for k in range(_K):
    sl = slice(k * _EC, (k + 1) * _EC)
    gs, gr = _sc_gather(snf, rnf, snd[sl], rcv[sl])
    raw, out_edge = _edge_mlp(k, gs, gr, ef[sl], raw, out_edge, *ew)

  zeros = jnp.zeros((_NS, _ZR, D), jnp.float32)
  partials = _sc_scatter(raw, rcv.reshape(_NW, _NSTR, _S),
                         zeros).reshape(_NC, N, D)

  new_node = _node_mlp(
      rnf, partials[0], partials[1],
      Wn0[0:D], Wn0[D:2 * D], bn0.reshape(1, D),
      Wn1, bn1.reshape(1, D), Wn2, bn2.reshape(1, D),
      lnn_g.reshape(1, D), lnn_b.reshape(1, D))

  return out_edge[None], new_node[None]


# speedup vs baseline: 1.4557x; 1.0303x over previous
import functools

import jax
import jax.numpy as jnp
from jax import lax
from jax.experimental import pallas as pl
from jax.experimental.pallas import tpu as pltpu
from jax.experimental.pallas import tpu_sc as plsc

N = 10000
E = 320000
D = 128

_NC = 2
_NS = 16
_NW = _NC * _NS
_R = E // _NW
_S = 80
_NSTR = _R // _S
_SLOTS = 5
_ZR = N // _NS
_SLOTS2 = 3
_K = 1
_EC = E // _K
_RC = _R // _K
_NSTRC = _RC // _S

@functools.cache
def _mesh():
  return plsc.VectorSubcoreMesh(core_axis_name="c", subcore_axis_name="s",
                                num_cores=_NC, num_subcores=_NS)


_GSLOTS = 3


def _gather_body(ts_hbm, tr_hbm, idxs_hbm, idxr_hbm, out_s, out_r,
                 idx_v, buf, gsem, tab_sh):
  cid = lax.axis_index("c")
  sid = lax.axis_index("s")
  wid = sid * _NC + cid
  row0 = wid * _RC
  zr8 = (_ZR // 8) * 8

  main = _NSTRC - (_NSTRC % _GSLOTS)

  for tab_hbm, idx_hbm, out in ((ts_hbm, idxs_hbm, out_s),
                                (tr_hbm, idxr_hbm, out_r)):
    pltpu.sync_copy(tab_hbm.at[pl.ds(sid * zr8, zr8)],
                    tab_sh.at[pl.ds(sid * zr8, zr8)])

    @pl.when(sid == 0)
    def _():
      rem = N - _NS * zr8
      pltpu.sync_copy(tab_hbm.at[pl.ds(_NS * zr8, rem)],
                      tab_sh.at[pl.ds(_NS * zr8, rem)])

    pltpu.sync_copy(idx_hbm.at[pl.ds(row0, _RC)], idx_v)
    plsc.subcore_barrier()

    def fire(g, b):
      pltpu.async_copy(tab_sh.at[idx_v.at[pl.ds(g * _S, _S)]], buf.at[b],
                       gsem.at[b])

    def consume(g, b):
      pltpu.make_async_copy(tab_sh.at[idx_v.at[pl.ds(g * _S, _S)]],
                            buf.at[b], gsem.at[b]).wait()
      pltpu.sync_copy(buf.at[b], out.at[pl.ds(row0 + g * _S, _S)])

    for b in range(_GSLOTS):
      fire(b, b)

    @pl.loop(0, main, step=_GSLOTS)
    def _(g0):
      for b in range(_GSLOTS):
        g = g0 + b
        consume(g, b)

        @pl.when(g + _GSLOTS < _NSTRC)
        def _():
          fire(g + _GSLOTS, b)

    for t in range(main, _NSTRC):
      consume(t, t - main)

    plsc.subcore_barrier()


def _sc_gather(snf, rnf, snd, rcv):
  return pl.kernel(
      _gather_body,
      out_type=(jax.ShapeDtypeStruct((_EC, D), jnp.float32),
                jax.ShapeDtypeStruct((_EC, D), jnp.float32)),
      mesh=_mesh(),
      scratch_types=[
          pltpu.VMEM((_RC,), jnp.int32),
          pltpu.VMEM((_GSLOTS, _S, D), jnp.float32),
          pltpu.SemaphoreType.DMA((_GSLOTS,)),
          pltpu.VMEM_SHARED((N, D), jnp.float32),
      ],
  )(snf, rnf, snd, rcv)


def _scatter_body(edge_hbm, idx_hbm, zeros_hbm, out_hbm,
                  idx_v, buf, lsem, acc_sh):
  cid = lax.axis_index("c")
  sid = lax.axis_index("s")
  wid = sid * _NC + cid
  row0 = wid * _R

  pltpu.sync_copy(zeros_hbm.at[sid], acc_sh.at[pl.ds(sid * _ZR, _ZR)])
  pltpu.sync_copy(idx_hbm.at[wid], idx_v)
  plsc.subcore_barrier()

  def fire(g, b):
    pltpu.async_copy(edge_hbm.at[pl.ds(row0 + g * _S, _S)], buf.at[b],
                     lsem.at[b])

  def consume(g, b):
    pltpu.make_async_copy(edge_hbm.at[pl.ds(row0 + g * _S, _S)],
                          buf.at[b], lsem.at[b]).wait()
    pltpu.sync_copy(buf.at[b], acc_sh.at[idx_v.at[g]], add=True)

  for b in range(_SLOTS2):
    fire(b, b)

  main = _NSTR - (_NSTR % _SLOTS2)

  @pl.loop(0, main, step=_SLOTS2)
  def _(g0):
    for b in range(_SLOTS2):
      g = g0 + b
      consume(g, b)

      @pl.when(g + _SLOTS2 < _NSTR)
      def _():
        fire(g + _SLOTS2, b)

  for t in range(main, _NSTR):
    consume(t, t - main)

  plsc.subcore_barrier()
  pltpu.sync_copy(acc_sh.at[pl.ds(sid * _ZR, _ZR)], out_hbm.at[cid, sid])


def _sc_scatter(raw, rcv, zeros):
  return pl.kernel(
      _scatter_body,
      out_type=jax.ShapeDtypeStruct((_NC, _NS, _ZR, D), jnp.float32),
      mesh=_mesh(),
      scratch_types=[
          pltpu.VMEM((_NSTR, _S), jnp.int32),
          pltpu.VMEM((_SLOTS2, _S, D), jnp.float32),
          pltpu.SemaphoreType.DMA((_SLOTS2,)),
          pltpu.VMEM_SHARED((N, D), jnp.float32),
      ],
  )(raw, rcv, zeros)


_EB = 8000


def _edge_mlp_compute(gs, gr, ef, w0s, w0r, w0e, b0, w1, b1, w2, b2,
                      ln_g, ln_b, raw, oe):
  f32 = jnp.float32
  e = ef[...]
  x = (jnp.dot(gs[...], w0s[...], preferred_element_type=f32)
       + jnp.dot(gr[...], w0r[...], preferred_element_type=f32)
       + jnp.dot(e, w0e[...], preferred_element_type=f32) + b0[...])
  x = jnp.maximum(x, 0.0)
  x = jnp.maximum(jnp.dot(x, w1[...], preferred_element_type=f32) + b1[...],
                  0.0)
  x = jnp.dot(x, w2[...], preferred_element_type=f32) + b2[...]
  mu = jnp.mean(x, axis=1, keepdims=True)
  xc = x - mu
  var = jnp.mean(xc * xc, axis=1, keepdims=True)
  y = ln_g[...] * (xc * lax.rsqrt(var + 1e-5)) + ln_b[...]
  raw[...] = y
  oe[...] = y + e


def _edge_mlp_body0(gs, gr, ef, w0s, w0r, w0e, b0, w1, b1, w2, b2,
                    ln_g, ln_b, raw, oe):
  _edge_mlp_compute(gs, gr, ef, w0s, w0r, w0e, b0, w1, b1, w2, b2,
                    ln_g, ln_b, raw, oe)


def _edge_mlp_bodyk(gs, gr, ef, w0s, w0r, w0e, b0, w1, b1, w2, b2,
                    ln_g, ln_b, raw_in, oe_in, raw, oe):
  del raw_in, oe_in
  _edge_mlp_compute(gs, gr, ef, w0s, w0r, w0e, b0, w1, b1, w2, b2,
                    ln_g, ln_b, raw, oe)


def _edge_mlp(k, gs, gr, ef, raw_buf, oe_buf, w0s, w0r, w0e, b0, w1, b1, w2,
              b2, ln_g, ln_b):
  koff = k * (_EC // _EB)
  crow = pl.BlockSpec((_EB, D), lambda i: (i, 0))
  orow = pl.BlockSpec((_EB, D), lambda i, koff=koff: (koff + i, 0))
  mat = pl.BlockSpec((D, D), lambda i: (0, 0))
  vec = pl.BlockSpec((1, D), lambda i: (0, 0))
  any_ = pl.BlockSpec(memory_space=pl.ANY)
  wspecs = [mat, mat, mat, vec, mat, vec, mat, vec, vec, vec]
  out_shape = (jax.ShapeDtypeStruct((E, D), jnp.float32),
               jax.ShapeDtypeStruct((E, D), jnp.float32))
  args = (gs, gr, ef) + (w0s, w0r, w0e, b0, w1, b1, w2, b2, ln_g, ln_b)
  if k == 0:
    return pl.pallas_call(
        _edge_mlp_body0,
        grid=(_EC // _EB,),
        in_specs=[crow, crow, crow] + wspecs,
        out_specs=[orow, orow],
        out_shape=out_shape,
    )(*args)
  return pl.pallas_call(
      _edge_mlp_bodyk,
      grid=(_EC // _EB,),
      in_specs=[crow, crow, crow] + wspecs + [any_, any_],
      out_specs=[orow, orow],
      out_shape=out_shape,
      input_output_aliases={13: 0, 14: 1},
  )(*args, raw_buf, oe_buf)


_NB = 2000


def _node_mlp_body(rnf, p0, p1, w0r, w0a, b0, w1, b1, w2, b2, ln_g, ln_b,
                   out):
  f32 = jnp.float32
  r = rnf[...]
  acc = p0[...] + p1[...]
  x = (jnp.dot(r, w0r[...], preferred_element_type=f32)
       + jnp.dot(acc, w0a[...], preferred_element_type=f32) + b0[...])
  x = jnp.maximum(x, 0.0)
  x = jnp.maximum(jnp.dot(x, w1[...], preferred_element_type=f32) + b1[...],
                  0.0)
  x = jnp.dot(x, w2[...], preferred_element_type=f32) + b2[...]
  mu = jnp.mean(x, axis=1, keepdims=True)
  xc = x - mu
  var = jnp.mean(xc * xc, axis=1, keepdims=True)
  out[...] = ln_g[...] * (xc * lax.rsqrt(var + 1e-5)) + ln_b[...] + r


def _node_mlp(rnf, p0, p1, w0r, w0a, b0, w1, b1, w2, b2, ln_g, ln_b):
  row = pl.BlockSpec((_NB, D), lambda i: (i, 0))
  mat = pl.BlockSpec((D, D), lambda i: (0, 0))
  vec = pl.BlockSpec((1, D), lambda i: (0, 0))
  return pl.pallas_call(
      _node_mlp_body,
      grid=(N // _NB,),
      in_specs=[row, row, row, mat, mat, vec, mat, vec, mat, vec, vec, vec],
      out_specs=row,
      out_shape=jax.ShapeDtypeStruct((N, D), jnp.float32),
  )(rnf, p0, p1, w0r, w0a, b0, w1, b1, w2, b2, ln_g, ln_b)


def kernel(senders, receivers, edge_features, sender_node_features,
           receiver_node_features, We0, be0, We1, be1, We2, be2, lne_g, lne_b,
           Wn0, bn0, Wn1, bn1, Wn2, bn2, lnn_g, lnn_b):
  snd = senders[0].astype(jnp.int32)
  rcv = receivers[0].astype(jnp.int32)
  ef = edge_features[0]
  snf = sender_node_features[0]
  rnf = receiver_node_features[0]
  ew = (We0[0:D], We0[D:2 * D], We0[2 * D:3 * D], be0.reshape(1, D),
        We1, be1.reshape(1, D), We2, be2.reshape(1, D),
        lne_g.reshape(1, D), lne_b.reshape(1, D))

  raw, out_edge = None, None
  for k in range(_K):
    sl = slice(k * _EC, (k + 1) * _EC)
    gs, gr = _sc_gather(snf, rnf, snd[sl], rcv[sl])
    raw, out_edge = _edge_mlp(k, gs, gr, ef[sl], raw, out_edge, *ew)

  zeros = jnp.zeros((_NS, _ZR, D), jnp.float32)
  partials = _sc_scatter(raw, rcv.reshape(_NW, _NSTR, _S),
                         zeros).reshape(_NC, N, D)

  new_node = _node_mlp(
      rnf, partials[0], partials[1],
      Wn0[0:D], Wn0[D:2 * D], bn0.reshape(1, D),
      Wn1, bn1.reshape(1, D), Wn2, bn2.reshape(1, D),
      lnn_g.reshape(1, D), lnn_b.reshape(1, D))

  return out_edge[None], new_node[None]


# speedup vs baseline: 1.4579x; 1.0015x over previous
import functools

import jax
import jax.numpy as jnp
from jax import lax
from jax.experimental import pallas as pl
from jax.experimental.pallas import tpu as pltpu
from jax.experimental.pallas import tpu_sc as plsc

N = 10000
E = 320000
D = 128

_NC = 2
_NS = 16
_NW = _NC * _NS
_R = E // _NW
_S = 80
_NSTR = _R // _S
_SLOTS = 5
_ZR = N // _NS
_SLOTS2 = 3
_K = 1
_EC = E // _K
_RC = _R // _K
_NSTRC = _RC // _S

@functools.cache
def _mesh():
  return plsc.VectorSubcoreMesh(core_axis_name="c", subcore_axis_name="s",
                                num_cores=_NC, num_subcores=_NS)


_GSLOTS = 3


def _gather_body(ts_hbm, tr_hbm, idxs_hbm, idxr_hbm, out_s, out_r,
                 idx_v, buf, gsem, tab_sh):
  cid = lax.axis_index("c")
  sid = lax.axis_index("s")
  wid = sid * _NC + cid
  row0 = wid * _RC
  zr8 = (_ZR // 8) * 8

  main = _NSTRC - (_NSTRC % _GSLOTS)

  for tab_hbm, idx_hbm, out in ((ts_hbm, idxs_hbm, out_s),
                                (tr_hbm, idxr_hbm, out_r)):
    pltpu.sync_copy(tab_hbm.at[pl.ds(sid * zr8, zr8)],
                    tab_sh.at[pl.ds(sid * zr8, zr8)])

    @pl.when(sid == 0)
    def _():
      rem = N - _NS * zr8
      pltpu.sync_copy(tab_hbm.at[pl.ds(_NS * zr8, rem)],
                      tab_sh.at[pl.ds(_NS * zr8, rem)])

    pltpu.sync_copy(idx_hbm.at[pl.ds(row0, _RC)], idx_v)
    plsc.subcore_barrier()

    def fire(g, b):
      pltpu.async_copy(tab_sh.at[idx_v.at[pl.ds(g * _S, _S)]], buf.at[b],
                       gsem.at[b])

    def consume(g, b):
      pltpu.make_async_copy(tab_sh.at[idx_v.at[pl.ds(g * _S, _S)]],
                            buf.at[b], gsem.at[b]).wait()
      pltpu.sync_copy(buf.at[b], out.at[pl.ds(row0 + g * _S, _S)])

    for b in range(_GSLOTS):
      fire(b, b)

    @pl.loop(0, main, step=_GSLOTS)
    def _(g0):
      for b in range(_GSLOTS):
        g = g0 + b
        consume(g, b)

        @pl.when(g + _GSLOTS < _NSTRC)
        def _():
          fire(g + _GSLOTS, b)

    for t in range(main, _NSTRC):
      consume(t, t - main)

    plsc.subcore_barrier()


def _sc_gather(snf, rnf, snd, rcv):
  return pl.kernel(
      _gather_body,
      out_type=(jax.ShapeDtypeStruct((_EC, D), jnp.float32),
                jax.ShapeDtypeStruct((_EC, D), jnp.float32)),
      mesh=_mesh(),
      scratch_types=[
          pltpu.VMEM((_RC,), jnp.int32),
          pltpu.VMEM((_GSLOTS, _S, D), jnp.float32),
          pltpu.SemaphoreType.DMA((_GSLOTS,)),
          pltpu.VMEM_SHARED((N, D), jnp.float32),
      ],
  )(snf, rnf, snd, rcv)


def _scatter_body(edge_hbm, idx_hbm, zeros_hbm, out_hbm,
                  idx_v, buf, lsem, acc_sh):
  cid = lax.axis_index("c")
  sid = lax.axis_index("s")
  wid = sid * _NC + cid
  row0 = wid * _R

  pltpu.sync_copy(zeros_hbm.at[sid], acc_sh.at[pl.ds(sid * _ZR, _ZR)])
  pltpu.sync_copy(idx_hbm.at[wid], idx_v)
  plsc.subcore_barrier()

  def fire(g, b):
    pltpu.async_copy(edge_hbm.at[pl.ds(row0 + g * _S, _S)], buf.at[b],
                     lsem.at[b])

  def consume(g, b):
    pltpu.make_async_copy(edge_hbm.at[pl.ds(row0 + g * _S, _S)],
                          buf.at[b], lsem.at[b]).wait()
    pltpu.sync_copy(buf.at[b], acc_sh.at[idx_v.at[g]], add=True)

  for b in range(_SLOTS2):
    fire(b, b)

  main = _NSTR - (_NSTR % _SLOTS2)

  @pl.loop(0, main, step=_SLOTS2)
  def _(g0):
    for b in range(_SLOTS2):
      g = g0 + b
      consume(g, b)

      @pl.when(g + _SLOTS2 < _NSTR)
      def _():
        fire(g + _SLOTS2, b)

  for t in range(main, _NSTR):
    consume(t, t - main)

  plsc.subcore_barrier()
  pltpu.sync_copy(acc_sh.at[pl.ds(sid * _ZR, _ZR)], out_hbm.at[cid, sid])


def _sc_scatter(raw, rcv, zeros):
  return pl.kernel(
      _scatter_body,
      out_type=jax.ShapeDtypeStruct((_NC, _NS, _ZR, D), jnp.float32),
      mesh=_mesh(),
      scratch_types=[
          pltpu.VMEM((_NSTR, _S), jnp.int32),
          pltpu.VMEM((_SLOTS2, _S, D), jnp.float32),
          pltpu.SemaphoreType.DMA((_SLOTS2,)),
          pltpu.VMEM_SHARED((N, D), jnp.float32),
      ],
  )(raw, rcv, zeros)


_EB = 10000


def _edge_mlp_compute(gs, gr, ef, w0s, w0r, w0e, b0, w1, b1, w2, b2,
                      ln_g, ln_b, raw, oe):
  f32 = jnp.float32
  e = ef[...]
  x = (jnp.dot(gs[...], w0s[...], preferred_element_type=f32)
       + jnp.dot(gr[...], w0r[...], preferred_element_type=f32)
       + jnp.dot(e, w0e[...], preferred_element_type=f32) + b0[...])
  x = jnp.maximum(x, 0.0)
  x = jnp.maximum(jnp.dot(x, w1[...], preferred_element_type=f32) + b1[...],
                  0.0)
  x = jnp.dot(x, w2[...], preferred_element_type=f32) + b2[...]
  mu = jnp.mean(x, axis=1, keepdims=True)
  xc = x - mu
  var = jnp.mean(xc * xc, axis=1, keepdims=True)
  y = ln_g[...] * (xc * lax.rsqrt(var + 1e-5)) + ln_b[...]
  raw[...] = y
  oe[...] = y + e


def _edge_mlp_body0(gs, gr, ef, w0s, w0r, w0e, b0, w1, b1, w2, b2,
                    ln_g, ln_b, raw, oe):
  _edge_mlp_compute(gs, gr, ef, w0s, w0r, w0e, b0, w1, b1, w2, b2,
                    ln_g, ln_b, raw, oe)


def _edge_mlp_bodyk(gs, gr, ef, w0s, w0r, w0e, b0, w1, b1, w2, b2,
                    ln_g, ln_b, raw_in, oe_in, raw, oe):
  del raw_in, oe_in
  _edge_mlp_compute(gs, gr, ef, w0s, w0r, w0e, b0, w1, b1, w2, b2,
                    ln_g, ln_b, raw, oe)


def _edge_mlp(k, gs, gr, ef, raw_buf, oe_buf, w0s, w0r, w0e, b0, w1, b1, w2,
              b2, ln_g, ln_b):
  koff = k * (_EC // _EB)
  crow = pl.BlockSpec((_EB, D), lambda i: (i, 0))
  orow = pl.BlockSpec((_EB, D), lambda i, koff=koff: (koff + i, 0))
  mat = pl.BlockSpec((D, D), lambda i: (0, 0))
  vec = pl.BlockSpec((1, D), lambda i: (0, 0))
  any_ = pl.BlockSpec(memory_space=pl.ANY)
  wspecs = [mat, mat, mat, vec, mat, vec, mat, vec, vec, vec]
  out_shape = (jax.ShapeDtypeStruct((E, D), jnp.float32),
               jax.ShapeDtypeStruct((E, D), jnp.float32))
  args = (gs, gr, ef) + (w0s, w0r, w0e, b0, w1, b1, w2, b2, ln_g, ln_b)
  if k == 0:
    return pl.pallas_call(
        _edge_mlp_body0,
        grid=(_EC // _EB,),
        in_specs=[crow, crow, crow] + wspecs,
        out_specs=[orow, orow],
        out_shape=out_shape,
    )(*args)
  return pl.pallas_call(
      _edge_mlp_bodyk,
      grid=(_EC // _EB,),
      in_specs=[crow, crow, crow] + wspecs + [any_, any_],
      out_specs=[orow, orow],
      out_shape=out_shape,
      input_output_aliases={13: 0, 14: 1},
  )(*args, raw_buf, oe_buf)


_NB = 5000


def _node_mlp_body(rnf, p0, p1, w0r, w0a, b0, w1, b1, w2, b2, ln_g, ln_b,
                   out):
  f32 = jnp.float32
  r = rnf[...]
  acc = p0[...] + p1[...]
  x = (jnp.dot(r, w0r[...], preferred_element_type=f32)
       + jnp.dot(acc, w0a[...], preferred_element_type=f32) + b0[...])
  x = jnp.maximum(x, 0.0)
  x = jnp.maximum(jnp.dot(x, w1[...], preferred_element_type=f32) + b1[...],
                  0.0)
  x = jnp.dot(x, w2[...], preferred_element_type=f32) + b2[...]
  mu = jnp.mean(x, axis=1, keepdims=True)
  xc = x - mu
  var = jnp.mean(xc * xc, axis=1, keepdims=True)
  out[...] = ln_g[...] * (xc * lax.rsqrt(var + 1e-5)) + ln_b[...] + r


def _node_mlp(rnf, p0, p1, w0r, w0a, b0, w1, b1, w2, b2, ln_g, ln_b):
  row = pl.BlockSpec((_NB, D), lambda i: (i, 0))
  mat = pl.BlockSpec((D, D), lambda i: (0, 0))
  vec = pl.BlockSpec((1, D), lambda i: (0, 0))
  return pl.pallas_call(
      _node_mlp_body,
      grid=(N // _NB,),
      in_specs=[row, row, row, mat, mat, vec, mat, vec, mat, vec, vec, vec],
      out_specs=row,
      out_shape=jax.ShapeDtypeStruct((N, D), jnp.float32),
  )(rnf, p0, p1, w0r, w0a, b0, w1, b1, w2, b2, ln_g, ln_b)


def kernel(senders, receivers, edge_features, sender_node_features,
           receiver_node_features, We0, be0, We1, be1, We2, be2, lne_g, lne_b,
           Wn0, bn0, Wn1, bn1, Wn2, bn2, lnn_g, lnn_b):
  snd = senders[0].astype(jnp.int32)
  rcv = receivers[0].astype(jnp.int32)
  ef = edge_features[0]
  snf = sender_node_features[0]
  rnf = receiver_node_features[0]
  ew = (We0[0:D], We0[D:2 * D], We0[2 * D:3 * D], be0.reshape(1, D),
        We1, be1.reshape(1, D), We2, be2.reshape(1, D),
        lne_g.reshape(1, D), lne_b.reshape(1, D))

  raw, out_edge = None, None
  for k in range(_K):
    sl = slice(k * _EC, (k + 1) * _EC)
    gs, gr = _sc_gather(snf, rnf, snd[sl], rcv[sl])
    raw, out_edge = _edge_mlp(k, gs, gr, ef[sl], raw, out_edge, *ew)

  zeros = jnp.zeros((_NS, _ZR, D), jnp.float32)
  partials = _sc_scatter(raw, rcv.reshape(_NW, _NSTR, _S),
                         zeros).reshape(_NC, N, D)

  new_node = _node_mlp(
      rnf, partials[0], partials[1],
      Wn0[0:D], Wn0[D:2 * D], bn0.reshape(1, D),
      Wn1, bn1.reshape(1, D), Wn2, bn2.reshape(1, D),
      lnn_g.reshape(1, D), lnn_b.reshape(1, D))

  return out_edge[None], new_node[None]


# speedup vs baseline: 1.4631x; 1.0036x over previous
import functools

import jax
import jax.numpy as jnp
from jax import lax
from jax.experimental import pallas as pl
from jax.experimental.pallas import tpu as pltpu
from jax.experimental.pallas import tpu_sc as plsc

N = 10000
E = 320000
D = 128

_NC = 2
_NS = 16
_NW = _NC * _NS
_R = E // _NW
_S = 80
_NSTR = _R // _S
_SLOTS = 5
_ZR = N // _NS
_SLOTS2 = 3
_K = 1
_EC = E // _K
_RC = _R // _K
_NSTRC = _RC // _S

@functools.cache
def _mesh():
  return plsc.VectorSubcoreMesh(core_axis_name="c", subcore_axis_name="s",
                                num_cores=_NC, num_subcores=_NS)


_GSLOTS = 3


def _gather_body(ts_hbm, tr_hbm, idxs_hbm, idxr_hbm, out_s, out_r,
                 idx_v, buf, gsem, tab_sh):
  cid = lax.axis_index("c")
  sid = lax.axis_index("s")
  wid = sid * _NC + cid
  row0 = wid * _RC
  zr8 = (_ZR // 8) * 8

  main = _NSTRC - (_NSTRC % _GSLOTS)

  for tab_hbm, idx_hbm, out in ((ts_hbm, idxs_hbm, out_s),
                                (tr_hbm, idxr_hbm, out_r)):
    pltpu.sync_copy(tab_hbm.at[pl.ds(sid * zr8, zr8)],
                    tab_sh.at[pl.ds(sid * zr8, zr8)])

    @pl.when(sid == 0)
    def _():
      rem = N - _NS * zr8
      pltpu.sync_copy(tab_hbm.at[pl.ds(_NS * zr8, rem)],
                      tab_sh.at[pl.ds(_NS * zr8, rem)])

    pltpu.sync_copy(idx_hbm.at[pl.ds(row0, _RC)], idx_v)
    plsc.subcore_barrier()

    def fire(g, b):
      pltpu.async_copy(tab_sh.at[idx_v.at[pl.ds(g * _S, _S)]], buf.at[b],
                       gsem.at[b])

    def consume(g, b):
      pltpu.make_async_copy(tab_sh.at[idx_v.at[pl.ds(g * _S, _S)]],
                            buf.at[b], gsem.at[b]).wait()
      pltpu.sync_copy(buf.at[b], out.at[pl.ds(row0 + g * _S, _S)])

    for b in range(_GSLOTS):
      fire(b, b)

    @pl.loop(0, main, step=_GSLOTS)
    def _(g0):
      for b in range(_GSLOTS):
        g = g0 + b
        consume(g, b)

        @pl.when(g + _GSLOTS < _NSTRC)
        def _():
          fire(g + _GSLOTS, b)

    for t in range(main, _NSTRC):
      consume(t, t - main)

    plsc.subcore_barrier()


def _sc_gather(snf, rnf, snd, rcv):
  return pl.kernel(
      _gather_body,
      out_type=(jax.ShapeDtypeStruct((_EC, D), jnp.float32),
                jax.ShapeDtypeStruct((_EC, D), jnp.float32)),
      mesh=_mesh(),
      scratch_types=[
          pltpu.VMEM((_RC,), jnp.int32),
          pltpu.VMEM((_GSLOTS, _S, D), jnp.float32),
          pltpu.SemaphoreType.DMA((_GSLOTS,)),
          pltpu.VMEM_SHARED((N, D), jnp.float32),
      ],
  )(snf, rnf, snd, rcv)


def _scatter_body(edge_hbm, idx_hbm, zeros_hbm, out_hbm,
                  idx_v, buf, lsem, acc_sh):
  cid = lax.axis_index("c")
  sid = lax.axis_index("s")
  wid = sid * _NC + cid
  row0 = wid * _R

  pltpu.sync_copy(zeros_hbm.at[sid], acc_sh.at[pl.ds(sid * _ZR, _ZR)])
  pltpu.sync_copy(idx_hbm.at[wid], idx_v)
  plsc.subcore_barrier()

  def fire(g, b):
    pltpu.async_copy(edge_hbm.at[pl.ds(row0 + g * _S, _S)], buf.at[b],
                     lsem.at[b])

  def consume(g, b):
    pltpu.make_async_copy(edge_hbm.at[pl.ds(row0 + g * _S, _S)],
                          buf.at[b], lsem.at[b]).wait()
    pltpu.sync_copy(buf.at[b], acc_sh.at[idx_v.at[g]], add=True)

  for b in range(_SLOTS2):
    fire(b, b)

  main = _NSTR - (_NSTR % _SLOTS2)

  @pl.loop(0, main, step=_SLOTS2)
  def _(g0):
    for b in range(_SLOTS2):
      g = g0 + b
      consume(g, b)

      @pl.when(g + _SLOTS2 < _NSTR)
      def _():
        fire(g + _SLOTS2, b)

  for t in range(main, _NSTR):
    consume(t, t - main)

  plsc.subcore_barrier()
  pltpu.sync_copy(acc_sh.at[pl.ds(sid * _ZR, _ZR)], out_hbm.at[cid, sid])


def _sc_scatter(raw, rcv, zeros):
  return pl.kernel(
      _scatter_body,
      out_type=jax.ShapeDtypeStruct((_NC, _NS, _ZR, D), jnp.float32),
      mesh=_mesh(),
      scratch_types=[
          pltpu.VMEM((_NSTR, _S), jnp.int32),
          pltpu.VMEM((_SLOTS2, _S, D), jnp.float32),
          pltpu.SemaphoreType.DMA((_SLOTS2,)),
          pltpu.VMEM_SHARED((N, D), jnp.float32),
      ],
  )(raw, rcv, zeros)


_EB = 10000


def _edge_mlp_compute(gs, gr, ef, w0s, w0r, w0e, b0, w1, b1, w2, b2,
                      ln_g, ln_b, raw, oe):
  f32 = jnp.float32
  bf16 = jnp.bfloat16
  e = ef[...]
  x = (jnp.dot(gs[...].astype(bf16), w0s[...], preferred_element_type=f32)
       + jnp.dot(gr[...].astype(bf16), w0r[...], preferred_element_type=f32)
       + jnp.dot(e.astype(bf16), w0e[...], preferred_element_type=f32)
       + b0[...])
  x = jnp.maximum(x, 0.0)
  x = jnp.maximum(
      jnp.dot(x.astype(bf16), w1[...], preferred_element_type=f32) + b1[...],
      0.0)
  x = jnp.dot(x.astype(bf16), w2[...], preferred_element_type=f32) + b2[...]
  mu = jnp.mean(x, axis=1, keepdims=True)
  xc = x - mu
  var = jnp.mean(xc * xc, axis=1, keepdims=True)
  y = ln_g[...] * (xc * lax.rsqrt(var + 1e-5)) + ln_b[...]
  raw[...] = y
  oe[...] = y + e


def _edge_mlp_body0(gs, gr, ef, w0s, w0r, w0e, b0, w1, b1, w2, b2,
                    ln_g, ln_b, raw, oe):
  _edge_mlp_compute(gs, gr, ef, w0s, w0r, w0e, b0, w1, b1, w2, b2,
                    ln_g, ln_b, raw, oe)


def _edge_mlp_bodyk(gs, gr, ef, w0s, w0r, w0e, b0, w1, b1, w2, b2,
                    ln_g, ln_b, raw_in, oe_in, raw, oe):
  del raw_in, oe_in
  _edge_mlp_compute(gs, gr, ef, w0s, w0r, w0e, b0, w1, b1, w2, b2,
                    ln_g, ln_b, raw, oe)


def _edge_mlp(k, gs, gr, ef, raw_buf, oe_buf, w0s, w0r, w0e, b0, w1, b1, w2,
              b2, ln_g, ln_b):
  koff = k * (_EC // _EB)
  crow = pl.BlockSpec((_EB, D), lambda i: (i, 0))
  orow = pl.BlockSpec((_EB, D), lambda i, koff=koff: (koff + i, 0))
  mat = pl.BlockSpec((D, D), lambda i: (0, 0))
  vec = pl.BlockSpec((1, D), lambda i: (0, 0))
  any_ = pl.BlockSpec(memory_space=pl.ANY)
  wspecs = [mat, mat, mat, vec, mat, vec, mat, vec, vec, vec]
  out_shape = (jax.ShapeDtypeStruct((E, D), jnp.float32),
               jax.ShapeDtypeStruct((E, D), jnp.float32))
  args = (gs, gr, ef) + (w0s, w0r, w0e, b0, w1, b1, w2, b2, ln_g, ln_b)
  if k == 0:
    return pl.pallas_call(
        _edge_mlp_body0,
        grid=(_EC // _EB,),
        in_specs=[crow, crow, crow] + wspecs,
        out_specs=[orow, orow],
        out_shape=out_shape,
    )(*args)
  return pl.pallas_call(
      _edge_mlp_bodyk,
      grid=(_EC // _EB,),
      in_specs=[crow, crow, crow] + wspecs + [any_, any_],
      out_specs=[orow, orow],
      out_shape=out_shape,
      input_output_aliases={13: 0, 14: 1},
  )(*args, raw_buf, oe_buf)


_NB = 5000


def _node_mlp_body(rnf, p0, p1, w0r, w0a, b0, w1, b1, w2, b2, ln_g, ln_b,
                   out):
  f32 = jnp.float32
  bf16 = jnp.bfloat16
  r = rnf[...]
  acc = p0[...] + p1[...]
  x = (jnp.dot(r.astype(bf16), w0r[...], preferred_element_type=f32)
       + jnp.dot(acc.astype(bf16), w0a[...], preferred_element_type=f32)
       + b0[...])
  x = jnp.maximum(x, 0.0)
  x = jnp.maximum(
      jnp.dot(x.astype(bf16), w1[...], preferred_element_type=f32) + b1[...],
      0.0)
  x = jnp.dot(x.astype(bf16), w2[...], preferred_element_type=f32) + b2[...]
  mu = jnp.mean(x, axis=1, keepdims=True)
  xc = x - mu
  var = jnp.mean(xc * xc, axis=1, keepdims=True)
  out[...] = ln_g[...] * (xc * lax.rsqrt(var + 1e-5)) + ln_b[...] + r


def _node_mlp(rnf, p0, p1, w0r, w0a, b0, w1, b1, w2, b2, ln_g, ln_b):
  row = pl.BlockSpec((_NB, D), lambda i: (i, 0))
  mat = pl.BlockSpec((D, D), lambda i: (0, 0))
  vec = pl.BlockSpec((1, D), lambda i: (0, 0))
  return pl.pallas_call(
      _node_mlp_body,
      grid=(N // _NB,),
      in_specs=[row, row, row, mat, mat, vec, mat, vec, mat, vec, vec, vec],
      out_specs=row,
      out_shape=jax.ShapeDtypeStruct((N, D), jnp.float32),
  )(rnf, p0, p1, w0r, w0a, b0, w1, b1, w2, b2, ln_g, ln_b)


def kernel(senders, receivers, edge_features, sender_node_features,
           receiver_node_features, We0, be0, We1, be1, We2, be2, lne_g, lne_b,
           Wn0, bn0, Wn1, bn1, Wn2, bn2, lnn_g, lnn_b):
  snd = senders[0].astype(jnp.int32)
  rcv = receivers[0].astype(jnp.int32)
  ef = edge_features[0]
  snf = sender_node_features[0]
  rnf = receiver_node_features[0]
  bf16 = jnp.bfloat16
  ew = (We0[0:D].astype(bf16), We0[D:2 * D].astype(bf16),
        We0[2 * D:3 * D].astype(bf16), be0.reshape(1, D),
        We1.astype(bf16), be1.reshape(1, D), We2.astype(bf16),
        be2.reshape(1, D), lne_g.reshape(1, D), lne_b.reshape(1, D))

  raw, out_edge = None, None
  for k in range(_K):
    sl = slice(k * _EC, (k + 1) * _EC)
    gs, gr = _sc_gather(snf, rnf, snd[sl], rcv[sl])
    raw, out_edge = _edge_mlp(k, gs, gr, ef[sl], raw, out_edge, *ew)

  zeros = jnp.zeros((_NS, _ZR, D), jnp.float32)
  partials = _sc_scatter(raw, rcv.reshape(_NW, _NSTR, _S),
                         zeros).reshape(_NC, N, D)

  new_node = _node_mlp(
      rnf, partials[0], partials[1],
      Wn0[0:D].astype(bf16), Wn0[D:2 * D].astype(bf16), bn0.reshape(1, D),
      Wn1.astype(bf16), bn1.reshape(1, D), Wn2.astype(bf16),
      bn2.reshape(1, D), lnn_g.reshape(1, D), lnn_b.reshape(1, D))

  return out_edge[None], new_node[None]
